# Initial kernel scaffold; baseline (speedup 1.0000x reference)
import jax
import jax.numpy as jnp
from jax.experimental import pallas as pl


def kernel(input, target, implication_filter_l, implication_filter_r):
    raise NotImplementedError("write your pallas kernel here")


# speedup vs baseline: 1.4380x; 1.4380x over previous
import functools

import jax
import jax.numpy as jnp
from jax import lax
from jax.experimental import pallas as pl
from jax.experimental.pallas import tpu as pltpu
from jax.experimental.pallas import tpu_sc as plsc

B = 4096
C = 1400
P = 10000

NC = 2
NS = 16
NW = NC * NS
ROWS_PER_W = B // NW
CHUNK_ROWS = 32
NCHUNK = ROWS_PER_W // CHUNK_ROWS
LANES = 16
PJ = P // LANES


def _tc_body(x_ref, t_ref, pred_ref, loss_ref):
    i = pl.program_id(0)
    x = x_ref[...]
    t = t_ref[...]
    pred_ref[...] = jax.nn.sigmoid(x)
    bce = jnp.maximum(x, 0.0) - x * t + jnp.log1p(jnp.exp(-jnp.abs(x)))
    s = jnp.sum(bce)

    @pl.when(i == 0)
    def _():
        loss_ref[0, 0] = 0.0

    loss_ref[0, 0] += s


def _sc_body(pred_hbm, l_hbm, r_hbm, out_hbm, chunk_v, l_v, r_v, res_v):
    c_id = lax.axis_index("c")
    s_id = lax.axis_index("s")
    wid = s_id * NC + c_id

    pltpu.sync_copy(l_hbm, l_v)
    pltpu.sync_copy(r_hbm, r_v)

    base = wid * (ROWS_PER_W * C)

    def chunk_loop(ci, accs):
        off = base + ci * (CHUNK_ROWS * C)
        pltpu.sync_copy(pred_hbm.at[pl.ds(off, CHUNK_ROWS * C)], chunk_v)

        def j_loop(j, accs):
            li = l_v[pl.ds(j * LANES, LANES)]
            ri = r_v[pl.ds(j * LANES, LANES)]
            a = list(accs)
            for rr in range(CHUNK_ROWS):
                lv = plsc.load_gather(chunk_v, [li + rr * C])
                rv = plsc.load_gather(chunk_v, [ri + rr * C])
                a[rr % 4] = a[rr % 4] + jnp.maximum(lv - rv, 0.0)
            return tuple(a)

        return lax.fori_loop(0, PJ, j_loop, accs)

    z = jnp.zeros((LANES,), jnp.float32)
    a0, a1, a2, a3 = lax.fori_loop(0, NCHUNK, chunk_loop, (z, z, z, z))
    res_v[...] = (a0 + a1) + (a2 + a3)
    pltpu.sync_copy(res_v, out_hbm.at[wid])


@functools.cache
def _get_sc_call():
    return pl.kernel(
        _sc_body,
        out_type=jax.ShapeDtypeStruct((NW, LANES), jnp.float32),
        mesh=plsc.VectorSubcoreMesh(core_axis_name="c", subcore_axis_name="s"),
        compiler_params=pltpu.CompilerParams(
            use_tc_tiling_on_sc=False, needs_layout_passes=False
        ),
        scratch_types=[
            pltpu.VMEM((CHUNK_ROWS * C,), jnp.float32),
            pltpu.VMEM((P,), jnp.int32),
            pltpu.VMEM((P,), jnp.int32),
            pltpu.VMEM((LANES,), jnp.float32),
        ],
    )


@jax.jit
def _run(x, t, l, r):
    pred, loss_sum = pl.pallas_call(
        _tc_body,
        grid=(NW,),
        in_specs=[
            pl.BlockSpec((ROWS_PER_W, C), lambda i: (i, 0)),
            pl.BlockSpec((ROWS_PER_W, C), lambda i: (i, 0)),
        ],
        out_specs=[
            pl.BlockSpec((ROWS_PER_W, C), lambda i: (i, 0)),
            pl.BlockSpec((1, 1), lambda i: (0, 0), memory_space=pltpu.SMEM),
        ],
        out_shape=[
            jax.ShapeDtypeStruct((B, C), jnp.float32),
            jax.ShapeDtypeStruct((1, 1), jnp.float32),
        ],
    )(x, t)

    partial = _get_sc_call()(pred.reshape(-1), l, r)
    base_loss = loss_sum[0, 0] / (B * C)
    implication_loss = jnp.sum(partial) / (B * P)
    return base_loss + implication_loss


def kernel(input, target, implication_filter_l, implication_filter_r):
    x = input.astype(jnp.float32)
    t = target.astype(jnp.float32)
    l = implication_filter_l.astype(jnp.int32)
    r = implication_filter_r.astype(jnp.int32)
    return _run(x, t, l, r)


# speedup vs baseline: 1.6642x; 1.1573x over previous
import functools

import jax
import jax.numpy as jnp
from jax import lax
from jax.experimental import pallas as pl
from jax.experimental.pallas import tpu as pltpu
from jax.experimental.pallas import tpu_sc as plsc

B = 4096
C = 1400
P = 10000

NC = 2
NS = 16
NW = NC * NS
LANES = 16
BLANES = 32
SLABS = 4
TC_GRID = 8
TC_BLK = B // TC_GRID


def _tc_body(xt_ref, tt_ref, predt_ref, loss_ref):
    i = pl.program_id(0)
    x = xt_ref[...]
    t = tt_ref[...]
    predt_ref[...] = jax.nn.sigmoid(x).astype(jnp.bfloat16)
    bce = jnp.maximum(x, 0.0) - x * t + jnp.log1p(jnp.exp(-jnp.abs(x)))
    s = jnp.sum(bce)

    @pl.when(i == 0)
    def _():
        loss_ref[0, 0] = 0.0

    loss_ref[0, 0] += s


def _sc_body(pred3_hbm, l_hbm, r_hbm, out_hbm, slab_v, l_v, r_v, res_v):
    c_id = lax.axis_index("c")
    s_id = lax.axis_index("s")
    wid = s_id * NC + c_id

    pltpu.sync_copy(l_hbm, l_v)
    pltpu.sync_copy(r_hbm, r_v)
    pltpu.sync_copy(pred3_hbm.at[:, pl.ds(wid * SLABS, SLABS), :], slab_v)

    def body(j, accs):
        lvec = l_v[pl.ds(j * LANES, LANES)]
        rvec = r_v[pl.ds(j * LANES, LANES)]
        a = list(accs)
        for k in range(LANES):
            li = lvec[k]
            ri = rvec[k]
            for s4 in range(SLABS):
                lw = slab_v[li, s4, :]
                rw = slab_v[ri, s4, :]
                d = jnp.maximum(lw - rw, 0)
                u0, u1 = plsc.unpack(d, format=plsc.PackFormat.INTERLEAVED)
                a[2 * s4] = a[2 * s4] + u0
                a[2 * s4 + 1] = a[2 * s4 + 1] + u1
        return tuple(a)

    z = jnp.zeros((LANES,), jnp.float32)
    accs = lax.fori_loop(0, P // LANES, body, (z,) * (2 * SLABS))
    r = accs[0]
    for k in range(1, 2 * SLABS):
        r = r + accs[k]
    res_v[...] = r
    pltpu.sync_copy(res_v, out_hbm.at[wid])


@functools.cache
def _get_sc_call():
    return pl.kernel(
        _sc_body,
        out_type=jax.ShapeDtypeStruct((NW, LANES), jnp.float32),
        mesh=plsc.VectorSubcoreMesh(core_axis_name="c", subcore_axis_name="s"),
        compiler_params=pltpu.CompilerParams(
            use_tc_tiling_on_sc=False, needs_layout_passes=False
        ),
        scratch_types=[
            pltpu.VMEM((C, SLABS, BLANES), jnp.bfloat16),
            pltpu.VMEM((P,), jnp.int32),
            pltpu.VMEM((P,), jnp.int32),
            pltpu.VMEM((LANES,), jnp.float32),
        ],
    )


@jax.jit
def _run(x, t, l, r):
    xt = x.T
    tt = t.T
    predt, loss_sum = pl.pallas_call(
        _tc_body,
        grid=(TC_GRID,),
        in_specs=[
            pl.BlockSpec((C, TC_BLK), lambda i: (0, i)),
            pl.BlockSpec((C, TC_BLK), lambda i: (0, i)),
        ],
        out_specs=[
            pl.BlockSpec((C, TC_BLK), lambda i: (0, i)),
            pl.BlockSpec((1, 1), lambda i: (0, 0), memory_space=pltpu.SMEM),
        ],
        out_shape=[
            jax.ShapeDtypeStruct((C, B), jnp.bfloat16),
            jax.ShapeDtypeStruct((1, 1), jnp.float32),
        ],
    )(xt, tt)

    pred3 = predt.reshape(C, B // BLANES, BLANES)
    partial = _get_sc_call()(pred3, l, r)
    base_loss = loss_sum[0, 0] / (B * C)
    implication_loss = jnp.sum(partial) / (B * P)
    return base_loss + implication_loss


def kernel(input, target, implication_filter_l, implication_filter_r):
    x = input.astype(jnp.float32)
    t = target.astype(jnp.float32)
    l = implication_filter_l.astype(jnp.int32)
    r = implication_filter_r.astype(jnp.int32)
    return _run(x, t, l, r)


# speedup vs baseline: 2.4689x; 1.4835x over previous
import functools

import jax
import jax.numpy as jnp
from jax import lax
from jax.experimental import pallas as pl
from jax.experimental.pallas import tpu as pltpu
from jax.experimental.pallas import tpu_sc as plsc

B = 4096
C = 1400
P = 10000

NC = 2
NS = 16
NW = NC * NS
LANES = 16
BLANES = 32
BPW = B // NW
TC_GRID = 8
TC_BLK = B // TC_GRID
SUB = TC_BLK // BPW
CHUNK_C = 100
NCHUNK = C // CHUNK_C


def _tc_body(xt_ref, tt_ref, pred_ref, loss_ref):
    i = pl.program_id(0)
    x = xt_ref[...]
    t = tt_ref[...]
    p = jax.nn.sigmoid(x)
    for k in range(SUB):
        pred_ref[k] = p[:, k * BPW:(k + 1) * BPW]
    bce = jnp.maximum(x, 0.0) - x * t + jnp.log1p(jnp.exp(-jnp.abs(x)))
    s = jnp.sum(bce)

    @pl.when(i == 0)
    def _():
        loss_ref[0, 0] = 0.0

    loss_ref[0, 0] += s


def _sc_body(pred_hbm, l_hbm, r_hbm, out_hbm, slab_v, chunk_v, l_v, r_v, res_v):
    c_id = lax.axis_index("c")
    s_id = lax.axis_index("s")
    wid = s_id * NC + c_id

    pltpu.sync_copy(l_hbm, l_v)
    pltpu.sync_copy(r_hbm, r_v)

    for ci in range(NCHUNK):
        pltpu.sync_copy(pred_hbm.at[wid, pl.ds(ci * CHUNK_C, CHUNK_C), :], chunk_v)

        def conv(rr, _):
            for k in range(BPW // BLANES):
                a = chunk_v[rr, pl.ds(k * BLANES, LANES)]
                b = chunk_v[rr, pl.ds(k * BLANES + LANES, LANES)]
                w = plsc.pack(a, b, format=plsc.PackFormat.INTERLEAVED)
                slab_v[ci * CHUNK_C + rr, pl.ds(k * BLANES, BLANES)] = w
            return 0

        lax.fori_loop(0, CHUNK_C, conv, 0)

    def body(j, accs):
        lvec = l_v[pl.ds(j * LANES, LANES)]
        rvec = r_v[pl.ds(j * LANES, LANES)]
        a = list(accs)
        for k in range(LANES):
            li = lvec[k]
            ri = rvec[k]
            for s4 in range(BPW // BLANES):
                lw = slab_v[li, pl.ds(s4 * BLANES, BLANES)]
                rw = slab_v[ri, pl.ds(s4 * BLANES, BLANES)]
                d = jnp.maximum(lw - rw, 0)
                u0, u1 = plsc.unpack(d, format=plsc.PackFormat.INTERLEAVED)
                a[2 * s4] = a[2 * s4] + u0
                a[2 * s4 + 1] = a[2 * s4 + 1] + u1
        return tuple(a)

    z = jnp.zeros((LANES,), jnp.float32)
    accs = lax.fori_loop(0, P // LANES, body, (z,) * 8)
    r = accs[0]
    for k in range(1, 8):
        r = r + accs[k]
    res_v[...] = r
    pltpu.sync_copy(res_v, out_hbm.at[wid])


@functools.cache
def _get_sc_call():
    return pl.kernel(
        _sc_body,
        out_type=jax.ShapeDtypeStruct((NW, LANES), jnp.float32),
        mesh=plsc.VectorSubcoreMesh(core_axis_name="c", subcore_axis_name="s"),
        compiler_params=pltpu.CompilerParams(
            use_tc_tiling_on_sc=False, needs_layout_passes=False
        ),
        scratch_types=[
            pltpu.VMEM((C, BPW), jnp.bfloat16),
            pltpu.VMEM((CHUNK_C, BPW), jnp.float32),
            pltpu.VMEM((P,), jnp.int32),
            pltpu.VMEM((P,), jnp.int32),
            pltpu.VMEM((LANES,), jnp.float32),
        ],
    )


@jax.jit
def _run(x, t, l, r):
    xt = x.T
    tt = t.T
    pred, loss_sum = pl.pallas_call(
        _tc_body,
        grid=(TC_GRID,),
        in_specs=[
            pl.BlockSpec((C, TC_BLK), lambda i: (0, i)),
            pl.BlockSpec((C, TC_BLK), lambda i: (0, i)),
        ],
        out_specs=[
            pl.BlockSpec((SUB, C, BPW), lambda i: (i, 0, 0)),
            pl.BlockSpec((1, 1), lambda i: (0, 0), memory_space=pltpu.SMEM),
        ],
        out_shape=[
            jax.ShapeDtypeStruct((NW, C, BPW), jnp.float32),
            jax.ShapeDtypeStruct((1, 1), jnp.float32),
        ],
    )(xt, tt)

    partial = _get_sc_call()(pred, l, r)
    base_loss = loss_sum[0, 0] / (B * C)
    implication_loss = jnp.sum(partial) / (B * P)
    return base_loss + implication_loss


def kernel(input, target, implication_filter_l, implication_filter_r):
    x = input.astype(jnp.float32)
    t = target.astype(jnp.float32)
    l = implication_filter_l.astype(jnp.int32)
    r = implication_filter_r.astype(jnp.int32)
    return _run(x, t, l, r)


# speedup vs baseline: 2.7485x; 1.1133x over previous
import functools

import jax
import jax.numpy as jnp
from jax import lax
from jax.experimental import pallas as pl
from jax.experimental.pallas import tpu as pltpu
from jax.experimental.pallas import tpu_sc as plsc

B = 4096
C = 1400
P = 10000

NC = 2
NS = 16
NW = NC * NS
LANES = 16
BLANES = 32
BPW = B // NW
TC_GRID = 8
TC_BLK = B // TC_GRID
SUB = TC_BLK // BPW
CHUNK_C = 70
NCHUNK = C // CHUNK_C


def _tc_body(xt_ref, tt_ref, pred_ref, loss_ref):
    i = pl.program_id(0)
    x = xt_ref[...]
    t = tt_ref[...]
    p = jax.nn.sigmoid(x)
    for k in range(SUB):
        pred_ref[k] = p[:, k * BPW:(k + 1) * BPW]
    bce = jnp.maximum(x, 0.0) - x * t + jnp.log1p(jnp.exp(-jnp.abs(x)))
    s = jnp.sum(bce)

    @pl.when(i == 0)
    def _():
        loss_ref[0, 0] = 0.0

    loss_ref[0, 0] += s


def _sc_body(
    pred_hbm, l_hbm, r_hbm, out_hbm, slab_v, chunk_v, l_v, r_v, res_v,
    sem_a, sem_b, sem_i
):
    c_id = lax.axis_index("c")
    s_id = lax.axis_index("s")
    wid = s_id * NC + c_id

    idx_l = pltpu.async_copy(l_hbm, l_v, sem_i)
    idx_r = pltpu.async_copy(r_hbm, r_v, sem_i)

    sems = (sem_a, sem_b)
    copies = [None, None]
    copies[0] = pltpu.async_copy(
        pred_hbm.at[wid, pl.ds(0, CHUNK_C), :], chunk_v.at[0], sems[0]
    )
    for ci in range(NCHUNK):
        p = ci % 2
        if ci + 1 < NCHUNK:
            copies[1 - p] = pltpu.async_copy(
                pred_hbm.at[wid, pl.ds((ci + 1) * CHUNK_C, CHUNK_C), :],
                chunk_v.at[1 - p],
                sems[1 - p],
            )
        copies[p].wait()

        def conv(rr, _):
            for k in range(BPW // BLANES):
                a = chunk_v[p, rr, pl.ds(k * BLANES, LANES)]
                b = chunk_v[p, rr, pl.ds(k * BLANES + LANES, LANES)]
                w = plsc.pack(a, b, format=plsc.PackFormat.INTERLEAVED)
                slab_v[ci * CHUNK_C + rr, pl.ds(k * BLANES, BLANES)] = w
            return 0

        lax.fori_loop(0, CHUNK_C, conv, 0)

    idx_l.wait()
    idx_r.wait()

    def body(j, accs):
        lvec = l_v[pl.ds(j * LANES, LANES)]
        rvec = r_v[pl.ds(j * LANES, LANES)]
        a = list(accs)
        for k in range(LANES):
            li = lvec[k]
            ri = rvec[k]
            for s4 in range(BPW // BLANES):
                lw = slab_v[li, pl.ds(s4 * BLANES, BLANES)]
                rw = slab_v[ri, pl.ds(s4 * BLANES, BLANES)]
                d = jnp.maximum(lw - rw, 0)
                u0, u1 = plsc.unpack(d, format=plsc.PackFormat.INTERLEAVED)
                a[2 * s4] = a[2 * s4] + u0
                a[2 * s4 + 1] = a[2 * s4 + 1] + u1
        return tuple(a)

    z = jnp.zeros((LANES,), jnp.float32)
    accs = lax.fori_loop(0, P // LANES, body, (z,) * 8)
    r = accs[0]
    for k in range(1, 8):
        r = r + accs[k]
    res_v[...] = r
    pltpu.sync_copy(res_v, out_hbm.at[wid])


@functools.cache
def _get_sc_call():
    return pl.kernel(
        _sc_body,
        out_type=jax.ShapeDtypeStruct((NW, LANES), jnp.float32),
        mesh=plsc.VectorSubcoreMesh(core_axis_name="c", subcore_axis_name="s"),
        compiler_params=pltpu.CompilerParams(
            use_tc_tiling_on_sc=False, needs_layout_passes=False
        ),
        scratch_types=[
            pltpu.VMEM((C, BPW), jnp.bfloat16),
            pltpu.VMEM((2, CHUNK_C, BPW), jnp.float32),
            pltpu.VMEM((P,), jnp.int32),
            pltpu.VMEM((P,), jnp.int32),
            pltpu.VMEM((LANES,), jnp.float32),
            pltpu.SemaphoreType.DMA,
            pltpu.SemaphoreType.DMA,
            pltpu.SemaphoreType.DMA,
        ],
    )


@jax.jit
def _run(x, t, l, r):
    xt = x.T
    tt = t.T
    pred, loss_sum = pl.pallas_call(
        _tc_body,
        grid=(TC_GRID,),
        in_specs=[
            pl.BlockSpec((C, TC_BLK), lambda i: (0, i)),
            pl.BlockSpec((C, TC_BLK), lambda i: (0, i)),
        ],
        out_specs=[
            pl.BlockSpec((SUB, C, BPW), lambda i: (i, 0, 0)),
            pl.BlockSpec((1, 1), lambda i: (0, 0), memory_space=pltpu.SMEM),
        ],
        out_shape=[
            jax.ShapeDtypeStruct((NW, C, BPW), jnp.float32),
            jax.ShapeDtypeStruct((1, 1), jnp.float32),
        ],
    )(xt, tt)

    partial = _get_sc_call()(pred, l, r)
    base_loss = loss_sum[0, 0] / (B * C)
    implication_loss = jnp.sum(partial) / (B * P)
    return base_loss + implication_loss


def kernel(input, target, implication_filter_l, implication_filter_r):
    x = input.astype(jnp.float32)
    t = target.astype(jnp.float32)
    l = implication_filter_l.astype(jnp.int32)
    r = implication_filter_r.astype(jnp.int32)
    return _run(x, t, l, r)
